# Initial kernel scaffold; baseline (speedup 1.0000x reference)
import jax
import jax.numpy as jnp
from jax.experimental import pallas as pl


def kernel(data, embed_weight):
    raise NotImplementedError("write your pallas kernel here")


# speedup vs baseline: 139.8817x; 139.8817x over previous
import jax
import jax.numpy as jnp
from jax import lax
from jax.experimental import pallas as pl

_C = 1000
_D = 16
_B = 1024
_BLK = 128


def _fill_kernel(d_ref, p_ref, o_ref):
    d = d_ref[...]
    col = lax.broadcasted_iota(jnp.int32, (_BLK, _C * _D), 1)
    slot = lax.shift_right_logical(col, 4)
    o_ref[...] = jnp.where(slot == d, p_ref[1:2, :], p_ref[0:1, :])


def kernel(data, embed_weight):
    pattern = jnp.broadcast_to(
        embed_weight[:2, None, :], (2, _C, _D)
    ).reshape(2, _C * _D)
    out = pl.pallas_call(
        _fill_kernel,
        grid=(_B // _BLK,),
        in_specs=[
            pl.BlockSpec((_BLK, 1), lambda i: (i, 0)),
            pl.BlockSpec((2, _C * _D), lambda i: (0, 0)),
        ],
        out_specs=pl.BlockSpec((_BLK, _C * _D), lambda i: (i, 0)),
        out_shape=jax.ShapeDtypeStruct((_B, _C * _D), jnp.float32),
    )(data.reshape(_B, 1), pattern)
    return out
